# baseline (speedup 1.0000x reference)
import jax
import jax.numpy as jnp
from jax.experimental import pallas as pl
from jax.experimental.pallas import tpu as pltpu

N = 4096
D = 20
BRA = 256
NBA = N // BRA
BRB = 512
NBB = N // BRB


def _conv1_kernel(s_ref, lap_ref, bd_ref, bu_ref, x_ref, xd_ref, xu_ref,
                  w0_ref, w1_ref, w2_ref, w3_ref, wq_ref, wk_ref, wv_ref,
                  q_ref, k_ref, v1_ref, m_ref):
    i = pl.program_id(0)
    x_blk = x_ref[pl.ds(i * BRA, BRA), :]
    acc = jnp.dot(x_blk, w0_ref[...], preferred_element_type=jnp.float32)
    acc += jnp.dot(jnp.dot(lap_ref[0], x_ref[...], preferred_element_type=jnp.float32),
                   w1_ref[...], preferred_element_type=jnp.float32)
    acc += jnp.dot(jnp.dot(bd_ref[0], xd_ref[...], preferred_element_type=jnp.float32),
                   w2_ref[...], preferred_element_type=jnp.float32)
    acc += jnp.dot(jnp.dot(bu_ref[0], xu_ref[...], preferred_element_type=jnp.float32),
                   w3_ref[...], preferred_element_type=jnp.float32)
    e1 = jnp.tanh(acc)
    q_ref[...] = jnp.dot(e1, wq_ref[...], preferred_element_type=jnp.float32)
    k_ref[...] = jnp.dot(e1, wk_ref[...], preferred_element_type=jnp.float32)
    v = jnp.dot(e1, wv_ref[...], preferred_element_type=jnp.float32)
    v1_ref[...] = jnp.concatenate([v, jnp.ones((BRA, 1), jnp.float32)], axis=1)
    m_ref[...] = (lap_ref[0] != 0.0).astype(jnp.int8)


def _attn_conv2_kernel(s_ref, m_ref, q_ref, k_ref, v1_ref,
                       lrow_ref, bdrow_ref, burow_ref, xd_ref, xu_ref,
                       w0_ref, w1_ref, w2_ref, w3_ref, lw_ref, lb_ref,
                       out_ref, acc_ref, e2row_ref):
    i = pl.program_id(0)
    idx = s_ref[3]
    scores = jax.lax.dot_general(q_ref[...], k_ref[...], (((1,), (1,)), ((), ())),
                                 preferred_element_type=jnp.float32)
    e = jnp.exp2(scores)
    w = e * m_ref[...].astype(jnp.float32)
    wv = jnp.dot(w, v1_ref[...], preferred_element_type=jnp.float32)
    e2_blk = wv[:, :D] / wv[:, D:D + 1]
    contrib = jnp.dot(lrow_ref[...], e2_blk, preferred_element_type=jnp.float32)
    prev = jnp.where(i == 0, jnp.zeros((1, D), jnp.float32), acc_ref[...])
    acc_ref[...] = prev + contrib

    @pl.when(i == idx // BRB)
    def _():
        rows = jax.lax.broadcasted_iota(jnp.int32, (BRB, 1), 0)
        sel = (rows == idx - i * BRB).astype(jnp.float32)
        e2row_ref[...] = jnp.sum(e2_blk * sel, axis=0, keepdims=True)

    @pl.when(i == NBB - 1)
    def _():
        r = jnp.dot(e2row_ref[...], w0_ref[...], preferred_element_type=jnp.float32)
        r += jnp.dot(acc_ref[...], w1_ref[...], preferred_element_type=jnp.float32)
        r += jnp.dot(jnp.dot(bdrow_ref[...], xd_ref[...], preferred_element_type=jnp.float32),
                     w2_ref[...], preferred_element_type=jnp.float32)
        r += jnp.dot(jnp.dot(burow_ref[...], xu_ref[...], preferred_element_type=jnp.float32),
                     w3_ref[...], preferred_element_type=jnp.float32)
        out_ref[...] = jnp.dot(jnp.tanh(r), lw_ref[...],
                               preferred_element_type=jnp.float32) + lb_ref[...]


def kernel(embeddings, laplacians, boundaries,
           conv1_W0, conv1_W1, conv1_W2, conv1_W3,
           attn_Wq, attn_Wk, attn_Wv,
           conv2_W0, conv2_W1, conv2_W2, conv2_W3,
           lin_W, lin_b, order, idx):
    f32 = jnp.float32
    order = jnp.asarray(order, jnp.int32)
    idx = jnp.asarray(idx, jnp.int32)
    od = jnp.clip(order - 1, 0, 3)
    ou = jnp.clip(order + 1, 0, 3)
    have_d = (order > 0).astype(f32)
    have_u = (order < 3).astype(f32)

    x = jax.lax.dynamic_index_in_dim(embeddings, order, 0, keepdims=False)
    xd = jax.lax.dynamic_index_in_dim(embeddings, od, 0, keepdims=False)
    xu = jax.lax.dynamic_index_in_dim(embeddings, ou, 0, keepdims=False)
    w2a = conv1_W2 * have_d
    w3a = conv1_W3 * have_u
    w2b = conv2_W2 * have_d
    w3b = conv2_W3 * have_u

    scalars = jnp.stack([order, od, ou, idx])

    def full2(shape):
        return pl.BlockSpec(shape, lambda i, s: (0, 0))

    def biga(which):
        return pl.BlockSpec((1, BRA, N), lambda i, s: (s[which], i, 0))

    q, k, v1, mask8 = pl.pallas_call(
        _conv1_kernel,
        grid_spec=pltpu.PrefetchScalarGridSpec(
            num_scalar_prefetch=1,
            grid=(NBA,),
            in_specs=[
                biga(0),
                biga(1),
                biga(2),
                full2((N, D)),
                full2((N, D)),
                full2((N, D)),
                full2((D, D)), full2((D, D)), full2((D, D)), full2((D, D)),
                full2((D, D)), full2((D, D)), full2((D, D)),
            ],
            out_specs=[
                pl.BlockSpec((BRA, D), lambda i, s: (i, 0)),
                pl.BlockSpec((BRA, D), lambda i, s: (i, 0)),
                pl.BlockSpec((BRA, D + 1), lambda i, s: (i, 0)),
                pl.BlockSpec((BRA, N), lambda i, s: (i, 0)),
            ],
        ),
        out_shape=[
            jax.ShapeDtypeStruct((N, D), f32),
            jax.ShapeDtypeStruct((N, D), f32),
            jax.ShapeDtypeStruct((N, D + 1), f32),
            jax.ShapeDtypeStruct((N, N), jnp.int8),
        ],
        compiler_params=pltpu.CompilerParams(vmem_limit_bytes=100 * 1024 * 1024),
    )(scalars, laplacians, boundaries, boundaries, x, xd, xu,
      conv1_W0, conv1_W1, w2a, w3a,
      attn_Wq * (1.4426950408889634 / (D ** 0.5)), attn_Wk, attn_Wv)

    lrow = jax.lax.dynamic_slice(laplacians, (order, idx, 0), (1, 1, N)).reshape(1, N)
    bdrow = jax.lax.dynamic_slice(boundaries, (od, idx, 0), (1, 1, N)).reshape(1, N)
    burow = jax.lax.dynamic_slice(boundaries, (ou, idx, 0), (1, 1, N)).reshape(1, N)

    out = pl.pallas_call(
        _attn_conv2_kernel,
        grid_spec=pltpu.PrefetchScalarGridSpec(
            num_scalar_prefetch=1,
            grid=(NBB,),
            in_specs=[
                pl.BlockSpec((BRB, N), lambda i, s: (i, 0)),
                pl.BlockSpec((BRB, D), lambda i, s: (i, 0)),
                full2((N, D)),
                full2((N, D + 1)),
                pl.BlockSpec((1, BRB), lambda i, s: (0, i)),
                full2((1, N)),
                full2((1, N)),
                full2((N, D)),
                full2((N, D)),
                full2((D, D)), full2((D, D)), full2((D, D)), full2((D, D)),
                full2((D, 10)), full2((1, 10)),
            ],
            out_specs=pl.BlockSpec((1, 10), lambda i, s: (0, 0)),
            scratch_shapes=[
                pltpu.VMEM((1, D), f32),
                pltpu.VMEM((1, D), f32),
            ],
        ),
        out_shape=jax.ShapeDtypeStruct((1, 10), f32),
        compiler_params=pltpu.CompilerParams(vmem_limit_bytes=100 * 1024 * 1024),
    )(scalars, mask8, q, k, v1, lrow, bdrow, burow, xd, xu,
      conv2_W0, conv2_W1, w2b, w3b, lin_W, lin_b.reshape(1, 10))

    return out.reshape(10)


# speedup vs baseline: 1.0057x; 1.0057x over previous
import jax
import jax.numpy as jnp
from jax.experimental import pallas as pl
from jax.experimental.pallas import tpu as pltpu

N = 4096
D = 20
BRA = 256
NBA = N // BRA
BRB = 1024
NBB = N // BRB


def _conv1_kernel(s_ref, lap_ref, bd_ref, bu_ref, x_ref, xd_ref, xu_ref,
                  w0_ref, w1_ref, w2_ref, w3_ref, wq_ref, wk_ref, wv_ref,
                  q_ref, k_ref, v1_ref, m_ref):
    i = pl.program_id(0)
    x_blk = x_ref[pl.ds(i * BRA, BRA), :]
    acc = jnp.dot(x_blk, w0_ref[...], preferred_element_type=jnp.float32)
    acc += jnp.dot(jnp.dot(lap_ref[0], x_ref[...], preferred_element_type=jnp.float32),
                   w1_ref[...], preferred_element_type=jnp.float32)
    acc += jnp.dot(jnp.dot(bd_ref[0], xd_ref[...], preferred_element_type=jnp.float32),
                   w2_ref[...], preferred_element_type=jnp.float32)
    acc += jnp.dot(jnp.dot(bu_ref[0], xu_ref[...], preferred_element_type=jnp.float32),
                   w3_ref[...], preferred_element_type=jnp.float32)
    e1 = jnp.tanh(acc)
    q_ref[...] = jnp.dot(e1, wq_ref[...], preferred_element_type=jnp.float32)
    k_ref[...] = jnp.dot(e1, wk_ref[...], preferred_element_type=jnp.float32)
    v = jnp.dot(e1, wv_ref[...], preferred_element_type=jnp.float32)
    v1_ref[...] = jnp.concatenate([v, jnp.ones((BRA, 1), jnp.float32)], axis=1)
    m_ref[...] = (lap_ref[0] != 0.0).astype(jnp.int8)


def _attn_conv2_kernel(s_ref, m_ref, q_ref, k_ref, v1_ref,
                       lrow_ref, bdrow_ref, burow_ref, xd_ref, xu_ref,
                       w0_ref, w1_ref, w2_ref, w3_ref, lw_ref, lb_ref,
                       out_ref, acc_ref, e2row_ref):
    i = pl.program_id(0)
    idx = s_ref[3]
    scores = jax.lax.dot_general(q_ref[...], k_ref[...], (((1,), (1,)), ((), ())),
                                 preferred_element_type=jnp.float32)
    e = jnp.exp2(scores)
    w = e * m_ref[...].astype(jnp.float32)
    wv = jnp.dot(w, v1_ref[...], preferred_element_type=jnp.float32)
    e2_blk = wv[:, :D] / wv[:, D:D + 1]
    contrib = jnp.dot(lrow_ref[...], e2_blk, preferred_element_type=jnp.float32)
    prev = jnp.where(i == 0, jnp.zeros((1, D), jnp.float32), acc_ref[...])
    acc_ref[...] = prev + contrib

    @pl.when(i == idx // BRB)
    def _():
        rows = jax.lax.broadcasted_iota(jnp.int32, (BRB, 1), 0)
        sel = (rows == idx - i * BRB).astype(jnp.float32)
        e2row_ref[...] = jnp.sum(e2_blk * sel, axis=0, keepdims=True)

    @pl.when(i == NBB - 1)
    def _():
        r = jnp.dot(e2row_ref[...], w0_ref[...], preferred_element_type=jnp.float32)
        r += jnp.dot(acc_ref[...], w1_ref[...], preferred_element_type=jnp.float32)
        r += jnp.dot(jnp.dot(bdrow_ref[...], xd_ref[...], preferred_element_type=jnp.float32),
                     w2_ref[...], preferred_element_type=jnp.float32)
        r += jnp.dot(jnp.dot(burow_ref[...], xu_ref[...], preferred_element_type=jnp.float32),
                     w3_ref[...], preferred_element_type=jnp.float32)
        out_ref[...] = jnp.dot(jnp.tanh(r), lw_ref[...],
                               preferred_element_type=jnp.float32) + lb_ref[...]


def kernel(embeddings, laplacians, boundaries,
           conv1_W0, conv1_W1, conv1_W2, conv1_W3,
           attn_Wq, attn_Wk, attn_Wv,
           conv2_W0, conv2_W1, conv2_W2, conv2_W3,
           lin_W, lin_b, order, idx):
    f32 = jnp.float32
    order = jnp.asarray(order, jnp.int32)
    idx = jnp.asarray(idx, jnp.int32)
    od = jnp.clip(order - 1, 0, 3)
    ou = jnp.clip(order + 1, 0, 3)
    have_d = (order > 0).astype(f32)
    have_u = (order < 3).astype(f32)

    x = jax.lax.dynamic_index_in_dim(embeddings, order, 0, keepdims=False)
    xd = jax.lax.dynamic_index_in_dim(embeddings, od, 0, keepdims=False)
    xu = jax.lax.dynamic_index_in_dim(embeddings, ou, 0, keepdims=False)
    w2a = conv1_W2 * have_d
    w3a = conv1_W3 * have_u
    w2b = conv2_W2 * have_d
    w3b = conv2_W3 * have_u

    scalars = jnp.stack([order, od, ou, idx])

    def full2(shape):
        return pl.BlockSpec(shape, lambda i, s: (0, 0))

    def biga(which):
        return pl.BlockSpec((1, BRA, N), lambda i, s: (s[which], i, 0))

    q, k, v1, mask8 = pl.pallas_call(
        _conv1_kernel,
        grid_spec=pltpu.PrefetchScalarGridSpec(
            num_scalar_prefetch=1,
            grid=(NBA,),
            in_specs=[
                biga(0),
                biga(1),
                biga(2),
                full2((N, D)),
                full2((N, D)),
                full2((N, D)),
                full2((D, D)), full2((D, D)), full2((D, D)), full2((D, D)),
                full2((D, D)), full2((D, D)), full2((D, D)),
            ],
            out_specs=[
                pl.BlockSpec((BRA, D), lambda i, s: (i, 0)),
                pl.BlockSpec((BRA, D), lambda i, s: (i, 0)),
                pl.BlockSpec((BRA, D + 1), lambda i, s: (i, 0)),
                pl.BlockSpec((BRA, N), lambda i, s: (i, 0)),
            ],
        ),
        out_shape=[
            jax.ShapeDtypeStruct((N, D), f32),
            jax.ShapeDtypeStruct((N, D), f32),
            jax.ShapeDtypeStruct((N, D + 1), f32),
            jax.ShapeDtypeStruct((N, N), jnp.int8),
        ],
        compiler_params=pltpu.CompilerParams(vmem_limit_bytes=100 * 1024 * 1024),
    )(scalars, laplacians, boundaries, boundaries, x, xd, xu,
      conv1_W0, conv1_W1, w2a, w3a,
      attn_Wq * (1.4426950408889634 / (D ** 0.5)), attn_Wk, attn_Wv)

    lrow = jax.lax.dynamic_slice(laplacians, (order, idx, 0), (1, 1, N)).reshape(1, N)
    bdrow = jax.lax.dynamic_slice(boundaries, (od, idx, 0), (1, 1, N)).reshape(1, N)
    burow = jax.lax.dynamic_slice(boundaries, (ou, idx, 0), (1, 1, N)).reshape(1, N)

    out = pl.pallas_call(
        _attn_conv2_kernel,
        grid_spec=pltpu.PrefetchScalarGridSpec(
            num_scalar_prefetch=1,
            grid=(NBB,),
            in_specs=[
                pl.BlockSpec((BRB, N), lambda i, s: (i, 0)),
                pl.BlockSpec((BRB, D), lambda i, s: (i, 0)),
                full2((N, D)),
                full2((N, D + 1)),
                pl.BlockSpec((1, BRB), lambda i, s: (0, i)),
                full2((1, N)),
                full2((1, N)),
                full2((N, D)),
                full2((N, D)),
                full2((D, D)), full2((D, D)), full2((D, D)), full2((D, D)),
                full2((D, 10)), full2((1, 10)),
            ],
            out_specs=pl.BlockSpec((1, 10), lambda i, s: (0, 0)),
            scratch_shapes=[
                pltpu.VMEM((1, D), f32),
                pltpu.VMEM((1, D), f32),
            ],
        ),
        out_shape=jax.ShapeDtypeStruct((1, 10), f32),
        compiler_params=pltpu.CompilerParams(vmem_limit_bytes=100 * 1024 * 1024),
    )(scalars, mask8, q, k, v1, lrow, bdrow, burow, xd, xu,
      conv2_W0, conv2_W1, w2b, w3b, lin_W, lin_b.reshape(1, 10))

    return out.reshape(10)
